# Initial kernel scaffold; baseline (speedup 1.0000x reference)
import functools

import jax
import jax.numpy as jnp
from jax import lax
from jax.experimental import pallas as pl
from jax.experimental.pallas import tpu as pltpu
from jax.experimental.pallas import tpu_sc as plsc

N = 10000
E = 160000
D = 128
DE = 16
G = 16
CF = 2 * D + DE
MD = CF // 2
MP = 144
FH = 2 * CF
GH = 2 * (D + MD)

NC = 2
NS = 16
NW = NC * NS
EPW = E // NW
GC = 128
GNC = EPW // GC
GT = EPW - GNC * GC
NPS = N // NS
ZCH = 125

_sc_mesh = plsc.VectorSubcoreMesh(
    core_axis_name="c", subcore_axis_name="s", num_cores=NC, num_subcores=NS)


@functools.partial(
    pl.kernel,
    out_type=(jax.ShapeDtypeStruct((E, D), jnp.float32),
              jax.ShapeDtypeStruct((E, D), jnp.float32)),
    mesh=_sc_mesh,
    scratch_types=(pltpu.VMEM((GC,), jnp.int32),
                   pltpu.VMEM((GC, D), jnp.float32),
                   pltpu.VMEM((GT,), jnp.int32),
                   pltpu.VMEM((GT, D), jnp.float32),
                   pltpu.SemaphoreType.DMA),
)
def _sc_gather(table, src, dst, xj, xi, idx_v, rows_v, idxt_v, rowst_v, sem):
    wid = lax.axis_index("s") * NC + lax.axis_index("c")
    base = wid * EPW

    def chunk(idx_hbm, out_hbm, off, iv, rv, n):
        pltpu.sync_copy(idx_hbm.at[pl.ds(off, n)], iv)
        pltpu.async_copy(table.at[iv], rv, sem).wait()
        pltpu.sync_copy(rv, out_hbm.at[pl.ds(off, n)])

    def body(k, carry):
        off = base + k * GC
        chunk(src, xj, off, idx_v, rows_v, GC)
        chunk(dst, xi, off, idx_v, rows_v, GC)
        return carry

    lax.fori_loop(0, GNC, body, 0)
    toff = base + GNC * GC
    chunk(src, xj, toff, idxt_v, rowst_v, GT)
    chunk(dst, xi, toff, idxt_v, rowst_v, GT)


@functools.partial(
    pl.kernel,
    out_type=jax.ShapeDtypeStruct((NC, N, MP), jnp.float32),
    mesh=_sc_mesh,
    scratch_types=(pltpu.VMEM((GC,), jnp.int32),
                   pltpu.VMEM((GC, MP), jnp.float32),
                   pltpu.VMEM((GT,), jnp.int32),
                   pltpu.VMEM((GT, MP), jnp.float32),
                   pltpu.VMEM((ZCH, MP), jnp.float32),
                   pltpu.VMEM_SHARED((N, MP), jnp.float32),
                   pltpu.SemaphoreType.DMA),
)
def _sc_scatter(msg, dsti, zstage, part, idx_v, msg_v, idxt_v, msgt_v,
                stage_v, acc, sem):
    c = lax.axis_index("c")
    s = lax.axis_index("s")
    wid = s * NC + c

    pltpu.sync_copy(zstage, stage_v)

    def zbody(j, carry):
        pltpu.sync_copy(stage_v, acc.at[pl.ds(s * NPS + j * ZCH, ZCH)])
        return carry

    lax.fori_loop(0, NPS // ZCH, zbody, 0)
    plsc.subcore_barrier()

    base = wid * EPW

    def chunk(off, iv, mv, n):
        pltpu.sync_copy(dsti.at[pl.ds(off, n)], iv)
        pltpu.sync_copy(msg.at[pl.ds(off, n)], mv)
        pltpu.sync_copy(mv, acc.at[iv], add=True)

    def body(k, carry):
        chunk(base + k * GC, idx_v, msg_v, GC)
        return carry

    lax.fori_loop(0, GNC, body, 0)
    chunk(base + GNC * GC, idxt_v, msgt_v, GT)
    plsc.subcore_barrier()

    def wbody(j, carry):
        r = s * NPS + j * ZCH
        pltpu.sync_copy(acc.at[pl.ds(r, ZCH)], stage_v)
        pltpu.sync_copy(stage_v, part.at[c, pl.ds(r, ZCH)])
        return carry

    lax.fori_loop(0, NPS // ZCH, wbody, 0)


TE = 2000
TN = 1000
ECH = 10000
_POOL_GRID = E // ECH


def _edge_body(xj, xi, ea,
               cw1a, cw1b, cw1c, cb1, cw2, cb2, cw3, cb3,
               pw1a, pw1b, pw1c, pb1, pw2, pb2, pw3, pb3,
               eo_ref, msg_ref):
    xjv = xj[...]
    xiv = xi[...]
    eav = ea[...]

    def mlp(w1a, w1b, w1c, b1, w2, b2, w3, b3):
        h = jnp.dot(xjv, w1a[...], preferred_element_type=jnp.float32)
        h = h + jnp.dot(xiv, w1b[...], preferred_element_type=jnp.float32)
        h = h + jnp.dot(eav, w1c[...], preferred_element_type=jnp.float32)
        h = jnp.maximum(h + b1[...], 0.0)
        h = jnp.dot(h, w2[...], preferred_element_type=jnp.float32) + b2[...]
        h = jnp.maximum(h, 0.0)
        return jnp.dot(h, w3[...], preferred_element_type=jnp.float32) + b3[...]

    eo_ref[...] = mlp(cw1a, cw1b, cw1c, cb1, cw2, cb2, cw3, cb3)
    msg_ref[...] = mlp(pw1a, pw1b, pw1c, pb1, pw2, pb2, pw3, pb3)


def _edge_call(xj, xi, ea, ws):
    row = lambda s: pl.BlockSpec((TE, s), lambda i: (i, 0))
    full = lambda a: pl.BlockSpec(a.shape, lambda i: tuple(0 for _ in a.shape))
    return pl.pallas_call(
        _edge_body,
        grid=(E // TE,),
        in_specs=[row(D), row(D), row(DE)] + [full(w) for w in ws],
        out_specs=[pl.BlockSpec((TE, DE), lambda i: (i, 0)),
                   pl.BlockSpec((TE, MP), lambda i: (i, 0))],
        out_shape=[jax.ShapeDtypeStruct((E, DE), jnp.float32),
                   jax.ShapeDtypeStruct((E, MP), jnp.float32)],
    )(xj, xi, ea, *ws)


def _gamma_body(p0, p1, xr, wa, wb, b1, w2, b2, w3, b3, out_ref):
    ssum = p0[...] + p1[...]
    colsel = (lax.broadcasted_iota(jnp.int32, (1, MP), 1) == MD)
    cnt = jnp.sum(jnp.where(colsel, ssum, 0.0), axis=1, keepdims=True)
    t = ssum / jnp.maximum(cnt, 1.0)
    h = jnp.dot(t, wa[...], preferred_element_type=jnp.float32)
    h = h + jnp.dot(xr[...], wb[...], preferred_element_type=jnp.float32)
    h = jnp.maximum(h + b1[...], 0.0)
    h = jnp.dot(h, w2[...], preferred_element_type=jnp.float32) + b2[...]
    h = jnp.maximum(h, 0.0)
    out_ref[...] = jnp.dot(h, w3[...], preferred_element_type=jnp.float32) + b3[...]


def _gamma_call(p0, p1, x, ws):
    row = lambda s: pl.BlockSpec((TN, s), lambda i: (i, 0))
    full = lambda a: pl.BlockSpec(a.shape, lambda i: tuple(0 for _ in a.shape))
    return pl.pallas_call(
        _gamma_body,
        grid=(N // TN,),
        in_specs=[row(MP), row(MP), row(D)] + [full(w) for w in ws],
        out_specs=pl.BlockSpec((TN, D), lambda i: (i, 0)),
        out_shape=jax.ShapeDtypeStruct((N, D), jnp.float32),
    )(p0, p1, x, *ws)


def _pool_body(e2_ref, eb_ref, x2_ref, nb_ref,
               an, ae, b1, w2, b2, w3, b3, out_ref, acc, cnt):
    i = pl.program_id(0)
    ids = lax.broadcasted_iota(jnp.int32, (G, ECH), 0)
    m = (eb_ref[0] == ids).astype(jnp.float32)
    part = jnp.dot(m, e2_ref[...], preferred_element_type=jnp.float32)
    c = jnp.sum(m, axis=1, keepdims=True)

    @pl.when(i == 0)
    def _():
        acc[...] = part
        cnt[...] = c

    @pl.when(i > 0)
    def _():
        acc[...] = acc[...] + part
        cnt[...] = cnt[...] + c

    @pl.when(i == _POOL_GRID - 1)
    def _():
        idn = lax.broadcasted_iota(jnp.int32, (G, N), 0)
        mn = (nb_ref[...] == idn).astype(jnp.float32)
        pn = jnp.dot(mn, x2_ref[...], preferred_element_type=jnp.float32)
        cn = jnp.sum(mn, axis=1, keepdims=True)
        pooled_n = pn / jnp.maximum(cn, 1.0)
        pooled_e = acc[...] / jnp.maximum(cnt[...], 1.0)
        h = jnp.dot(pooled_n, an[...], preferred_element_type=jnp.float32)
        h = h + jnp.dot(pooled_e, ae[...], preferred_element_type=jnp.float32)
        h = jnp.maximum(h + b1[...], 0.0)
        h = jnp.dot(h, w2[...], preferred_element_type=jnp.float32) + b2[...]
        h = jnp.maximum(h, 0.0)
        out_ref[...] = jnp.dot(h, w3[...], preferred_element_type=jnp.float32) + b3[...]


def _pool_call(e2, x2, nb2d, eb3d, ws):
    full = lambda a: pl.BlockSpec(a.shape, lambda i: tuple(0 for _ in a.shape))
    return pl.pallas_call(
        _pool_body,
        grid=(_POOL_GRID,),
        in_specs=[pl.BlockSpec((ECH, DE), lambda i: (i, 0)),
                  pl.BlockSpec((1, 1, ECH), lambda i: (i, 0, 0)),
                  full(x2), full(nb2d)] + [full(w) for w in ws],
        out_specs=pl.BlockSpec((G, 5), lambda i: (0, 0)),
        out_shape=jax.ShapeDtypeStruct((G, 5), jnp.float32),
        scratch_shapes=[pltpu.VMEM((G, DE), jnp.float32),
                        pltpu.VMEM((G, 1), jnp.float32)],
    )(e2, eb3d, x2, nb2d, *ws)


def _prep_edge(ps):
    (cw1, cb1), (cw2, cb2), (cw3, cb3) = ps['chi']
    (pw1, pb1), (pw2, pb2), (pw3, pb3) = ps['phi']
    pw3p = jnp.pad(pw3, ((0, 0), (0, MP - MD)))
    pb3p = jnp.pad(pb3, (0, MP - MD)).at[MD].set(1.0)
    return (cw1[:D], cw1[D:2 * D], cw1[2 * D:], cb1.reshape(1, -1),
            cw2, cb2.reshape(1, -1), cw3, cb3.reshape(1, -1),
            pw1[:D], pw1[D:2 * D], pw1[2 * D:], pb1.reshape(1, -1),
            pw2, pb2.reshape(1, -1), pw3p, pb3p.reshape(1, -1))


def _prep_gamma(ps):
    (gw1, gb1), (gw2, gb2), (gw3, gb3) = ps['gamma']
    wa = jnp.pad(gw1[:MD], ((0, MP - MD), (0, 0)))
    wb = gw1[MD:]
    return (wa, wb, gb1.reshape(1, -1), gw2, gb2.reshape(1, -1),
            gw3, gb3.reshape(1, -1))


def _prep_alpha(ps):
    (aw1, ab1), (aw2, ab2), (aw3, ab3) = ps
    return (aw1[:D], aw1[D:], ab1.reshape(1, -1), aw2, ab2.reshape(1, -1),
            aw3, ab3.reshape(1, -1))


def kernel(x, edge_index, edge_attr, node_batch, edge_batch, params):
    src = edge_index[0]
    dst = edge_index[1]
    zstage = jnp.zeros((ZCH, MP), jnp.float32)
    nb2d = node_batch.reshape(1, N)
    eb3d = edge_batch.reshape(_POOL_GRID, 1, ECH)

    ew1 = _prep_edge(params['layer1'])
    gw1 = _prep_gamma(params['layer1'])
    ew2 = _prep_edge(params['layer2'])
    gw2 = _prep_gamma(params['layer2'])
    aw = _prep_alpha(params['alpha'])

    xj, xi = _sc_gather(x, src, dst)
    e1, msg1 = _edge_call(xj, xi, edge_attr, ew1)
    part1 = _sc_scatter(msg1, dst, zstage)
    x1 = _gamma_call(part1[0], part1[1], x, gw1)

    xj2, xi2 = _sc_gather(x1, src, dst)
    e2, msg2 = _edge_call(xj2, xi2, e1, ew2)
    part2 = _sc_scatter(msg2, dst, zstage)
    x2 = _gamma_call(part2[0], part2[1], x1, gw2)

    coeff = _pool_call(e2, x2, nb2d, eb3d, aw)
    return (e2, x2, coeff)


# speedup vs baseline: 1.6427x; 1.6427x over previous
import functools

import jax
import jax.numpy as jnp
from jax import lax
from jax.experimental import pallas as pl
from jax.experimental.pallas import tpu as pltpu
from jax.experimental.pallas import tpu_sc as plsc

N = 10000
E = 160000
D = 128
DE = 16
G = 16
CF = 2 * D + DE
MD = CF // 2
MP = 144
FH = 2 * CF
GH = 2 * (D + MD)

NC = 2
NS = 16
NW = NC * NS
GC = 128
NP = 10240
NPS = NP // NS
ZCH = 128


@functools.cache
def _get_sc_gather(ne):
    epw = ne // NW
    gnc = epw // GC
    gt = epw - gnc * GC
    mesh = plsc.VectorSubcoreMesh(
        core_axis_name="c", subcore_axis_name="s",
        num_cores=NC, num_subcores=NS)

    def body_fn(table, src, dst, xj, xi, idx_v, rows_v, idxt_v, rowst_v, sem):
        wid = lax.axis_index("s") * NC + lax.axis_index("c")
        base = wid * epw

        def chunk(idx_hbm, out_hbm, off, iv, rv, n):
            pltpu.sync_copy(idx_hbm.at[pl.ds(off, n)], iv)
            pltpu.async_copy(table.at[iv], rv, sem).wait()
            pltpu.sync_copy(rv, out_hbm.at[pl.ds(off, n)])

        def body(k, carry):
            off = base + k * GC
            chunk(src, xj, off, idx_v, rows_v, GC)
            chunk(dst, xi, off, idx_v, rows_v, GC)
            return carry

        lax.fori_loop(0, gnc, body, 0)
        if gt:
            toff = base + gnc * GC
            chunk(src, xj, toff, idxt_v, rowst_v, gt)
            chunk(dst, xi, toff, idxt_v, rowst_v, gt)

    return functools.partial(
        pl.kernel,
        out_type=(jax.ShapeDtypeStruct((ne, D), jnp.float32),
                  jax.ShapeDtypeStruct((ne, D), jnp.float32)),
        mesh=mesh,
        scratch_types=(pltpu.VMEM((GC,), jnp.int32),
                       pltpu.VMEM((GC, D), jnp.float32),
                       pltpu.VMEM((max(gt, 8),), jnp.int32),
                       pltpu.VMEM((max(gt, 8), D), jnp.float32),
                       pltpu.SemaphoreType.DMA),
    )(body_fn)


@functools.cache
def _get_sc_scatter(ne):
    epw = ne // NW
    gnc = epw // GC
    gt = epw - gnc * GC
    mesh = plsc.VectorSubcoreMesh(
        core_axis_name="c", subcore_axis_name="s",
        num_cores=NC, num_subcores=NS)

    def body_fn(msg, dsti, zstage, part, idx_v, msg_v, idxt_v, msgt_v,
                stage_v, acc, sem):
        c = lax.axis_index("c")
        s = lax.axis_index("s")
        wid = s * NC + c

        pltpu.sync_copy(zstage, stage_v)

        def zbody(j, carry):
            pltpu.sync_copy(stage_v, acc.at[pl.ds(s * NPS + j * ZCH, ZCH)])
            return carry

        lax.fori_loop(0, NPS // ZCH, zbody, 0)
        plsc.subcore_barrier()

        base = wid * epw

        def chunk(off, iv, mv, n):
            c1 = pltpu.async_copy(dsti.at[pl.ds(off, n)], iv, sem)
            c2 = pltpu.async_copy(msg.at[pl.ds(off, n)], mv, sem)
            c1.wait()
            c2.wait()
            pltpu.sync_copy(mv, acc.at[iv], add=True)

        def body(k, carry):
            chunk(base + k * GC, idx_v, msg_v, GC)
            return carry

        lax.fori_loop(0, gnc, body, 0)
        if gt:
            chunk(base + gnc * GC, idxt_v, msgt_v, gt)
        plsc.subcore_barrier()

        def wbody(j, carry):
            r = s * NPS + j * ZCH
            pltpu.sync_copy(acc.at[pl.ds(r, ZCH)], stage_v)
            pltpu.sync_copy(stage_v, part.at[c, pl.ds(r, ZCH)])
            return carry

        lax.fori_loop(0, NPS // ZCH, wbody, 0)

    return functools.partial(
        pl.kernel,
        out_type=jax.ShapeDtypeStruct((NC, NP, MP), jnp.float32),
        mesh=mesh,
        compiler_params=pltpu.CompilerParams(use_tc_tiling_on_sc=False),
        scratch_types=(pltpu.VMEM((GC,), jnp.int32),
                       pltpu.VMEM((GC, MP), jnp.float32),
                       pltpu.VMEM((max(gt, 8),), jnp.int32),
                       pltpu.VMEM((max(gt, 8), MP), jnp.float32),
                       pltpu.VMEM((ZCH, MP), jnp.float32),
                       pltpu.VMEM_SHARED((NP, MP), jnp.float32),
                       pltpu.SemaphoreType.DMA),
    )(body_fn)


TN = 1000
ECH = 10000
_POOL_GRID = E // ECH


def _edge_body(xj, xi, ea,
               cw1a, cw1b, cw1c, cb1, cw2, cb2, cw3, cb3,
               pw1a, pw1b, pw1c, pb1, pw2, pb2, pw3, pb3,
               eo_ref, msg_ref):
    xjv = xj[...]
    xiv = xi[...]

    bf = jnp.bfloat16
    xjb = xjv.astype(bf)
    xib = xiv.astype(bf)

    def mlp(ev, w1a, w1b, w1c, b1, w2, b2, w3, b3):
        h = jnp.dot(xjb, w1a[...], preferred_element_type=jnp.float32)
        h = h + jnp.dot(xib, w1b[...], preferred_element_type=jnp.float32)
        h = h + jnp.dot(ev.astype(bf), w1c[...],
                        preferred_element_type=jnp.float32)
        h = jnp.maximum(h + b1[...], 0.0)
        h = jnp.dot(h.astype(bf), w2[...],
                    preferred_element_type=jnp.float32) + b2[...]
        h = jnp.maximum(h, 0.0)
        return jnp.dot(h.astype(bf), w3[...],
                       preferred_element_type=jnp.float32) + b3[...]

    eo = mlp(ea[...], cw1a, cw1b, cw1c, cb1, cw2, cb2, cw3, cb3)
    eo_ref[...] = eo
    msg_ref[...] = mlp(eo, pw1a, pw1b, pw1c, pb1, pw2, pb2, pw3, pb3)


def _edge_call(ne, te, xj, xi, ea, ws):
    row = lambda s: pl.BlockSpec((te, s), lambda i: (i, 0))
    full = lambda a: pl.BlockSpec(a.shape, lambda i: tuple(0 for _ in a.shape))
    return pl.pallas_call(
        _edge_body,
        grid=(ne // te,),
        in_specs=[row(D), row(D), row(DE)] + [full(w) for w in ws],
        out_specs=[pl.BlockSpec((te, DE), lambda i: (i, 0)),
                   pl.BlockSpec((te, MP), lambda i: (i, 0))],
        out_shape=[jax.ShapeDtypeStruct((ne, DE), jnp.float32),
                   jax.ShapeDtypeStruct((ne, MP), jnp.float32)],
    )(xj, xi, ea, *ws)


def _gamma_body(p0, p1, q0, q1, xr, wa, wb, b1, w2, b2, w3, b3, out_ref):
    ssum = p0[...] + p1[...] + q0[...] + q1[...]
    colsel = (lax.broadcasted_iota(jnp.int32, (1, MP), 1) == MD)
    cnt = jnp.sum(jnp.where(colsel, ssum, 0.0), axis=1, keepdims=True)
    t = ssum / jnp.maximum(cnt, 1.0)
    bf = jnp.bfloat16
    h = jnp.dot(t.astype(bf), wa[...], preferred_element_type=jnp.float32)
    h = h + jnp.dot(xr[...].astype(bf), wb[...],
                    preferred_element_type=jnp.float32)
    h = jnp.maximum(h + b1[...], 0.0)
    h = jnp.dot(h.astype(bf), w2[...],
                preferred_element_type=jnp.float32) + b2[...]
    h = jnp.maximum(h, 0.0)
    out_ref[...] = jnp.dot(h.astype(bf), w3[...],
                           preferred_element_type=jnp.float32) + b3[...]


def _gamma_call(parts, x, ws):
    row = lambda s: pl.BlockSpec((TN, s), lambda i: (i, 0))
    full = lambda a: pl.BlockSpec(a.shape, lambda i: tuple(0 for _ in a.shape))
    return pl.pallas_call(
        _gamma_body,
        grid=(N // TN,),
        in_specs=[row(MP), row(MP), row(MP), row(MP), row(D)]
        + [full(w) for w in ws],
        out_specs=pl.BlockSpec((TN, D), lambda i: (i, 0)),
        out_shape=jax.ShapeDtypeStruct((N, D), jnp.float32),
    )(*parts, x, *ws)


def _pool_body(e2_ref, eb_ref, x2_ref, nb_ref,
               an, ae, b1, w2, b2, w3, b3, out_ref, acc, cnt):
    i = pl.program_id(0)
    ids = lax.broadcasted_iota(jnp.int32, (G, ECH), 0)
    m = (eb_ref[0] == ids).astype(jnp.float32)
    part = jnp.dot(m, e2_ref[...], preferred_element_type=jnp.float32)
    c = jnp.sum(m, axis=1, keepdims=True)

    @pl.when(i == 0)
    def _():
        acc[...] = part
        cnt[...] = c

    @pl.when(i > 0)
    def _():
        acc[...] = acc[...] + part
        cnt[...] = cnt[...] + c

    @pl.when(i == _POOL_GRID - 1)
    def _():
        idn = lax.broadcasted_iota(jnp.int32, (G, N), 0)
        mn = (nb_ref[...] == idn).astype(jnp.float32)
        pn = jnp.dot(mn, x2_ref[...], preferred_element_type=jnp.float32)
        cn = jnp.sum(mn, axis=1, keepdims=True)
        pooled_n = pn / jnp.maximum(cn, 1.0)
        pooled_e = acc[...] / jnp.maximum(cnt[...], 1.0)
        h = jnp.dot(pooled_n, an[...], preferred_element_type=jnp.float32)
        h = h + jnp.dot(pooled_e, ae[...], preferred_element_type=jnp.float32)
        h = jnp.maximum(h + b1[...], 0.0)
        h = jnp.dot(h, w2[...], preferred_element_type=jnp.float32) + b2[...]
        h = jnp.maximum(h, 0.0)
        out_ref[...] = jnp.dot(h, w3[...], preferred_element_type=jnp.float32) + b3[...]


def _pool_call(e2, x2, nb2d, eb3d, ws):
    full = lambda a: pl.BlockSpec(a.shape, lambda i: tuple(0 for _ in a.shape))
    return pl.pallas_call(
        _pool_body,
        grid=(_POOL_GRID,),
        in_specs=[pl.BlockSpec((ECH, DE), lambda i: (i, 0)),
                  pl.BlockSpec((1, 1, ECH), lambda i: (i, 0, 0)),
                  full(x2), full(nb2d)] + [full(w) for w in ws],
        out_specs=pl.BlockSpec((G, 5), lambda i: (0, 0)),
        out_shape=jax.ShapeDtypeStruct((G, 5), jnp.float32),
        scratch_shapes=[pltpu.VMEM((G, DE), jnp.float32),
                        pltpu.VMEM((G, 1), jnp.float32)],
    )(e2, eb3d, x2, nb2d, *ws)


def _prep_edge(ps):
    (cw1, cb1), (cw2, cb2), (cw3, cb3) = ps['chi']
    (pw1, pb1), (pw2, pb2), (pw3, pb3) = ps['phi']
    bf = jnp.bfloat16
    pw3p = jnp.pad(pw3, ((0, 0), (0, MP - MD)))
    pb3p = jnp.pad(pb3, (0, MP - MD)).at[MD].set(1.0)
    return (cw1[:D].astype(bf), cw1[D:2 * D].astype(bf),
            cw1[2 * D:].astype(bf), cb1.reshape(1, -1),
            cw2.astype(bf), cb2.reshape(1, -1),
            cw3.astype(bf), cb3.reshape(1, -1),
            pw1[:D].astype(bf), pw1[D:2 * D].astype(bf),
            pw1[2 * D:].astype(bf), pb1.reshape(1, -1),
            pw2.astype(bf), pb2.reshape(1, -1),
            pw3p.astype(bf), pb3p.reshape(1, -1))


def _prep_gamma(ps):
    (gw1, gb1), (gw2, gb2), (gw3, gb3) = ps['gamma']
    bf = jnp.bfloat16
    wa = jnp.pad(gw1[:MD], ((0, MP - MD), (0, 0))).astype(bf)
    wb = gw1[MD:].astype(bf)
    return (wa, wb, gb1.reshape(1, -1), gw2.astype(bf), gb2.reshape(1, -1),
            gw3.astype(bf), gb3.reshape(1, -1))


def _prep_alpha(ps):
    (aw1, ab1), (aw2, ab2), (aw3, ab3) = ps
    return (aw1[:D], aw1[D:], ab1.reshape(1, -1), aw2, ab2.reshape(1, -1),
            aw3, ab3.reshape(1, -1))


E1 = 81920
E2 = E - E1
TE1 = 2048
TE2 = 2440


def _layer(src, dst, ea, ew, gw, xf, zstage):
    s1, s2 = src[:E1], src[E1:]
    d1, d2 = dst[:E1], dst[E1:]
    xj1, xi1 = _get_sc_gather(E1)(xf, s1, d1)
    xj2, xi2 = _get_sc_gather(E2)(xf, s2, d2)
    eo1, msg1 = _edge_call(E1, TE1, xj1, xi1, ea[:E1], ew)
    eo2, msg2 = _edge_call(E2, TE2, xj2, xi2, ea[E1:], ew)
    p1 = _get_sc_scatter(E1)(msg1, d1, zstage)
    p2 = _get_sc_scatter(E2)(msg2, d2, zstage)
    parts = (p1[0, :N], p1[1, :N], p2[0, :N], p2[1, :N])
    x_next = _gamma_call(parts, xf, gw)
    eo = jnp.concatenate([eo1, eo2], axis=0)
    return eo, x_next


def kernel(x, edge_index, edge_attr, node_batch, edge_batch, params):
    src = edge_index[0]
    dst = edge_index[1]
    zstage = jnp.zeros((ZCH, MP), jnp.float32)
    nb2d = node_batch.reshape(1, N)
    eb3d = edge_batch.reshape(_POOL_GRID, 1, ECH)

    ew1 = _prep_edge(params['layer1'])
    gw1 = _prep_gamma(params['layer1'])
    ew2 = _prep_edge(params['layer2'])
    gw2 = _prep_gamma(params['layer2'])
    aw = _prep_alpha(params['alpha'])

    e1, x1 = _layer(src, dst, edge_attr, ew1, gw1, x, zstage)
    e2, x2 = _layer(src, dst, e1, ew2, gw2, x1, zstage)

    coeff = _pool_call(e2, x2, nb2d, eb3d, aw)
    return (e2, x2, coeff)


# speedup vs baseline: 1.6691x; 1.0161x over previous
import functools

import jax
import jax.numpy as jnp
from jax import lax
from jax.experimental import pallas as pl
from jax.experimental.pallas import tpu as pltpu
from jax.experimental.pallas import tpu_sc as plsc

N = 10000
E = 160000
D = 128
DE = 16
G = 16
CF = 2 * D + DE
MD = CF // 2
MP = 144
FH = 2 * CF
GH = 2 * (D + MD)

NC = 2
NS = 16
NW = NC * NS
GC = 128
NP = 10240
NPS = NP // NS
ZCH = 128


@functools.cache
def _get_sc_gather(ne):
    epw = ne // NW
    gnc = epw // GC
    gt = epw - gnc * GC
    mesh = plsc.VectorSubcoreMesh(
        core_axis_name="c", subcore_axis_name="s",
        num_cores=NC, num_subcores=NS)

    def body_fn(table, src, dst, xj, xi, idx_v, rows_v, idxt_v, rowst_v, sem):
        wid = lax.axis_index("s") * NC + lax.axis_index("c")
        base = wid * epw

        def chunk(idx_hbm, out_hbm, off, iv, rv, n):
            pltpu.sync_copy(idx_hbm.at[pl.ds(off, n)], iv)
            pltpu.async_copy(table.at[iv], rv, sem).wait()
            pltpu.sync_copy(rv, out_hbm.at[pl.ds(off, n)])

        def body(k, carry):
            off = base + k * GC
            chunk(src, xj, off, idx_v, rows_v, GC)
            chunk(dst, xi, off, idx_v, rows_v, GC)
            return carry

        lax.fori_loop(0, gnc, body, 0)
        if gt:
            toff = base + gnc * GC
            chunk(src, xj, toff, idxt_v, rowst_v, gt)
            chunk(dst, xi, toff, idxt_v, rowst_v, gt)

    return functools.partial(
        pl.kernel,
        out_type=(jax.ShapeDtypeStruct((ne, D), jnp.float32),
                  jax.ShapeDtypeStruct((ne, D), jnp.float32)),
        mesh=mesh,
        scratch_types=(pltpu.VMEM((GC,), jnp.int32),
                       pltpu.VMEM((GC, D), jnp.float32),
                       pltpu.VMEM((max(gt, 8),), jnp.int32),
                       pltpu.VMEM((max(gt, 8), D), jnp.float32),
                       pltpu.SemaphoreType.DMA),
    )(body_fn)


@functools.cache
def _get_sc_scatter(ne):
    epw = ne // NW
    gnc = epw // GC
    gt = epw - gnc * GC
    mesh = plsc.VectorSubcoreMesh(
        core_axis_name="c", subcore_axis_name="s",
        num_cores=NC, num_subcores=NS)

    def body_fn(msg, dsti, zstage, part, idx_v, msg_v, idxt_v, msgt_v,
                stage_v, acc, sem):
        c = lax.axis_index("c")
        s = lax.axis_index("s")
        wid = s * NC + c

        pltpu.sync_copy(zstage, stage_v)

        def zbody(j, carry):
            pltpu.sync_copy(stage_v, acc.at[pl.ds(s * NPS + j * ZCH, ZCH)])
            return carry

        lax.fori_loop(0, NPS // ZCH, zbody, 0)
        plsc.subcore_barrier()

        base = wid * epw

        def chunk(off, iv, mv, n):
            c1 = pltpu.async_copy(dsti.at[pl.ds(off, n)], iv, sem)
            c2 = pltpu.async_copy(msg.at[pl.ds(off, n)], mv, sem)
            c1.wait()
            c2.wait()
            pltpu.sync_copy(mv, acc.at[iv], add=True)

        def body(k, carry):
            chunk(base + k * GC, idx_v, msg_v, GC)
            return carry

        lax.fori_loop(0, gnc, body, 0)
        if gt:
            chunk(base + gnc * GC, idxt_v, msgt_v, gt)
        plsc.subcore_barrier()

        def wbody(j, carry):
            r = s * NPS + j * ZCH
            pltpu.sync_copy(acc.at[pl.ds(r, ZCH)], stage_v)
            pltpu.sync_copy(stage_v, part.at[c, pl.ds(r, ZCH)])
            return carry

        lax.fori_loop(0, NPS // ZCH, wbody, 0)

    return functools.partial(
        pl.kernel,
        out_type=jax.ShapeDtypeStruct((NC, NP, MP), jnp.float32),
        mesh=mesh,
        compiler_params=pltpu.CompilerParams(use_tc_tiling_on_sc=False),
        scratch_types=(pltpu.VMEM((GC,), jnp.int32),
                       pltpu.VMEM((GC, MP), jnp.float32),
                       pltpu.VMEM((max(gt, 8),), jnp.int32),
                       pltpu.VMEM((max(gt, 8), MP), jnp.float32),
                       pltpu.VMEM((ZCH, MP), jnp.float32),
                       pltpu.VMEM_SHARED((NP, MP), jnp.float32),
                       pltpu.SemaphoreType.DMA),
    )(body_fn)


TN = 1000
ECH = 10000
_POOL_GRID = E // ECH


def _edge_body(xj, xi, ea,
               cw1a, cw1b, cw1c, cb1, cw2, cb2, cw3, cb3,
               pw1a, pw1b, pw1c, pb1, pw2, pb2, pw3, pb3,
               eo_ref, msg_ref):
    xjv = xj[...]
    xiv = xi[...]

    bf = jnp.bfloat16
    xjb = xjv.astype(bf)
    xib = xiv.astype(bf)

    def mlp(ev, w1a, w1b, w1c, b1, w2, b2, w3, b3):
        h = jnp.dot(xjb, w1a[...], preferred_element_type=jnp.float32)
        h = h + jnp.dot(xib, w1b[...], preferred_element_type=jnp.float32)
        h = h + jnp.dot(ev.astype(bf), w1c[...],
                        preferred_element_type=jnp.float32)
        h = jnp.maximum(h + b1[...], 0.0)
        h = jnp.dot(h.astype(bf), w2[...],
                    preferred_element_type=jnp.float32) + b2[...]
        h = jnp.maximum(h, 0.0)
        return jnp.dot(h.astype(bf), w3[...],
                       preferred_element_type=jnp.float32) + b3[...]

    eo = mlp(ea[...], cw1a, cw1b, cw1c, cb1, cw2, cb2, cw3, cb3)
    eo_ref[...] = eo
    msg_ref[...] = mlp(eo, pw1a, pw1b, pw1c, pb1, pw2, pb2, pw3, pb3)


def _edge_call(ne, te, xj, xi, ea, ws):
    row = lambda s: pl.BlockSpec((te, s), lambda i: (i, 0))
    full = lambda a: pl.BlockSpec(a.shape, lambda i: tuple(0 for _ in a.shape))
    return pl.pallas_call(
        _edge_body,
        grid=(ne // te,),
        in_specs=[row(D), row(D), row(DE)] + [full(w) for w in ws],
        out_specs=[pl.BlockSpec((te, DE), lambda i: (i, 0)),
                   pl.BlockSpec((te, MP), lambda i: (i, 0))],
        out_shape=[jax.ShapeDtypeStruct((ne, DE), jnp.float32),
                   jax.ShapeDtypeStruct((ne, MP), jnp.float32)],
    )(xj, xi, ea, *ws)


def _gamma_body(p0, p1, q0, q1, xr, wa, wb, b1, w2, b2, w3, b3, out_ref):
    ssum = p0[0] + p1[0] + q0[0] + q1[0]
    colsel = (lax.broadcasted_iota(jnp.int32, (1, MP), 1) == MD)
    cnt = jnp.sum(jnp.where(colsel, ssum, 0.0), axis=1, keepdims=True)
    t = ssum / jnp.maximum(cnt, 1.0)
    bf = jnp.bfloat16
    h = jnp.dot(t.astype(bf), wa[...], preferred_element_type=jnp.float32)
    h = h + jnp.dot(xr[...].astype(bf), wb[...],
                    preferred_element_type=jnp.float32)
    h = jnp.maximum(h + b1[...], 0.0)
    h = jnp.dot(h.astype(bf), w2[...],
                preferred_element_type=jnp.float32) + b2[...]
    h = jnp.maximum(h, 0.0)
    out_ref[...] = jnp.dot(h.astype(bf), w3[...],
                           preferred_element_type=jnp.float32) + b3[...]


def _gamma_call(p1, p2, x, ws):
    c0 = pl.BlockSpec((1, TN, MP), lambda i: (0, i, 0))
    c1 = pl.BlockSpec((1, TN, MP), lambda i: (1, i, 0))
    row = lambda s: pl.BlockSpec((TN, s), lambda i: (i, 0))
    full = lambda a: pl.BlockSpec(a.shape, lambda i: tuple(0 for _ in a.shape))
    return pl.pallas_call(
        _gamma_body,
        grid=(N // TN,),
        in_specs=[c0, c1, c0, c1, row(D)] + [full(w) for w in ws],
        out_specs=pl.BlockSpec((TN, D), lambda i: (i, 0)),
        out_shape=jax.ShapeDtypeStruct((N, D), jnp.float32),
    )(p1, p1, p2, p2, x, *ws)


def _pool_body(e2_ref, eb_ref, x2_ref, nb_ref,
               an, ae, b1, w2, b2, w3, b3, out_ref, acc, cnt):
    i = pl.program_id(0)
    ids = lax.broadcasted_iota(jnp.int32, (G, ECH), 0)
    m = (eb_ref[0] == ids).astype(jnp.float32)
    part = jnp.dot(m, e2_ref[...], preferred_element_type=jnp.float32)
    c = jnp.sum(m, axis=1, keepdims=True)

    @pl.when(i == 0)
    def _():
        acc[...] = part
        cnt[...] = c

    @pl.when(i > 0)
    def _():
        acc[...] = acc[...] + part
        cnt[...] = cnt[...] + c

    @pl.when(i == _POOL_GRID - 1)
    def _():
        idn = lax.broadcasted_iota(jnp.int32, (G, N), 0)
        mn = (nb_ref[...] == idn).astype(jnp.float32)
        pn = jnp.dot(mn, x2_ref[...], preferred_element_type=jnp.float32)
        cn = jnp.sum(mn, axis=1, keepdims=True)
        pooled_n = pn / jnp.maximum(cn, 1.0)
        pooled_e = acc[...] / jnp.maximum(cnt[...], 1.0)
        h = jnp.dot(pooled_n, an[...], preferred_element_type=jnp.float32)
        h = h + jnp.dot(pooled_e, ae[...], preferred_element_type=jnp.float32)
        h = jnp.maximum(h + b1[...], 0.0)
        h = jnp.dot(h, w2[...], preferred_element_type=jnp.float32) + b2[...]
        h = jnp.maximum(h, 0.0)
        out_ref[...] = jnp.dot(h, w3[...], preferred_element_type=jnp.float32) + b3[...]


def _pool_call(e2, x2, nb2d, eb3d, ws):
    full = lambda a: pl.BlockSpec(a.shape, lambda i: tuple(0 for _ in a.shape))
    return pl.pallas_call(
        _pool_body,
        grid=(_POOL_GRID,),
        in_specs=[pl.BlockSpec((ECH, DE), lambda i: (i, 0)),
                  pl.BlockSpec((1, 1, ECH), lambda i: (i, 0, 0)),
                  full(x2), full(nb2d)] + [full(w) for w in ws],
        out_specs=pl.BlockSpec((G, 5), lambda i: (0, 0)),
        out_shape=jax.ShapeDtypeStruct((G, 5), jnp.float32),
        scratch_shapes=[pltpu.VMEM((G, DE), jnp.float32),
                        pltpu.VMEM((G, 1), jnp.float32)],
    )(e2, eb3d, x2, nb2d, *ws)


def _prep_edge(ps):
    (cw1, cb1), (cw2, cb2), (cw3, cb3) = ps['chi']
    (pw1, pb1), (pw2, pb2), (pw3, pb3) = ps['phi']
    bf = jnp.bfloat16
    pw3p = jnp.pad(pw3, ((0, 0), (0, MP - MD)))
    pb3p = jnp.pad(pb3, (0, MP - MD)).at[MD].set(1.0)
    return (cw1[:D].astype(bf), cw1[D:2 * D].astype(bf),
            cw1[2 * D:].astype(bf), cb1.reshape(1, -1),
            cw2.astype(bf), cb2.reshape(1, -1),
            cw3.astype(bf), cb3.reshape(1, -1),
            pw1[:D].astype(bf), pw1[D:2 * D].astype(bf),
            pw1[2 * D:].astype(bf), pb1.reshape(1, -1),
            pw2.astype(bf), pb2.reshape(1, -1),
            pw3p.astype(bf), pb3p.reshape(1, -1))


def _prep_gamma(ps):
    (gw1, gb1), (gw2, gb2), (gw3, gb3) = ps['gamma']
    bf = jnp.bfloat16
    wa = jnp.pad(gw1[:MD], ((0, MP - MD), (0, 0))).astype(bf)
    wb = gw1[MD:].astype(bf)
    return (wa, wb, gb1.reshape(1, -1), gw2.astype(bf), gb2.reshape(1, -1),
            gw3.astype(bf), gb3.reshape(1, -1))


def _prep_alpha(ps):
    (aw1, ab1), (aw2, ab2), (aw3, ab3) = ps
    return (aw1[:D], aw1[D:], ab1.reshape(1, -1), aw2, ab2.reshape(1, -1),
            aw3, ab3.reshape(1, -1))


E1 = 81920
E2 = E - E1
TE1 = 2048
TE2 = 2440


def _layer(src, dst, ea1, ea2, ew, gw, xf, zstage):
    s1, s2 = src[:E1], src[E1:]
    d1, d2 = dst[:E1], dst[E1:]
    xj1, xi1 = _get_sc_gather(E1)(xf, s1, d1)
    xj2, xi2 = _get_sc_gather(E2)(xf, s2, d2)
    eo1, msg1 = _edge_call(E1, TE1, xj1, xi1, ea1, ew)
    eo2, msg2 = _edge_call(E2, TE2, xj2, xi2, ea2, ew)
    p1 = _get_sc_scatter(E1)(msg1, d1, zstage)
    p2 = _get_sc_scatter(E2)(msg2, d2, zstage)
    x_next = _gamma_call(p1, p2, xf, gw)
    return eo1, eo2, x_next


def kernel(x, edge_index, edge_attr, node_batch, edge_batch, params):
    src = edge_index[0]
    dst = edge_index[1]
    zstage = jnp.zeros((ZCH, MP), jnp.float32)
    nb2d = node_batch.reshape(1, N)
    eb3d = edge_batch.reshape(_POOL_GRID, 1, ECH)

    ew1 = _prep_edge(params['layer1'])
    gw1 = _prep_gamma(params['layer1'])
    ew2 = _prep_edge(params['layer2'])
    gw2 = _prep_gamma(params['layer2'])
    aw = _prep_alpha(params['alpha'])

    e1a, e1b, x1 = _layer(src, dst, edge_attr[:E1], edge_attr[E1:],
                          ew1, gw1, x, zstage)
    e2a, e2b, x2 = _layer(src, dst, e1a, e1b, ew2, gw2, x1, zstage)

    e2 = jnp.concatenate([e2a, e2b], axis=0)
    coeff = _pool_call(e2, x2, nb2d, eb3d, aw)
    return (e2, x2, coeff)
